# Initial kernel scaffold; baseline (speedup 1.0000x reference)
import jax
import jax.numpy as jnp
from jax.experimental import pallas as pl


def kernel(x, edge_index, batch, W1, b1, W2, b2, W3, b3, Wl, bl):
    raise NotImplementedError("write your pallas kernel here")


# speedup vs baseline: 35.4971x; 35.4971x over previous
import functools

import jax
import jax.numpy as jnp
from jax import lax
from jax.experimental import pallas as pl
from jax.experimental.pallas import tpu as pltpu
from jax.experimental.pallas import tpu_sc as plsc

N = 100000
E = 3200000
NUM_GRAPHS = 64
HIDDEN = 32
NUM_CLASSES = 3

NP = 100352
NROW = NP // 128
EP = 3211264
NC = 2
NS = 16
CH = 128
BCH = 8
BLK_E = CH * BCH
NPT = NP // NS
ER2D = EP // CH

_mesh = plsc.VectorSubcoreMesh(core_axis_name="c", subcore_axis_name="s")


def _fill_f32(buf, n16, value):
    def body(i, _):
        buf[pl.ds(i * 16, 16)] = jnp.full((16,), value, jnp.float32)
        return 0
    lax.fori_loop(0, n16, body, 0)


def _fill_rows_f32(buf, nrows, value):
    def body(i, _):
        buf[i, :] = jnp.full((16,), value, jnp.float32)
        return 0
    lax.fori_loop(0, nrows, body, 0)


@functools.partial(
    pl.kernel,
    out_type=jax.ShapeDtypeStruct((NC, NP), jnp.float32),
    mesh=_mesh,
    compiler_params=pltpu.CompilerParams(needs_layout_passes=False, use_tc_tiling_on_sc=False),
    scratch_types=[
        pltpu.VMEM((BCH, CH), jnp.int32),
        pltpu.VMEM((CH,), jnp.float32),
        pltpu.VMEM((NPT,), jnp.float32),
        pltpu.VMEM_SHARED((NP,), jnp.float32),
    ],
)
def _deg_kernel(dst2d_hbm, out_hbm, didx_blk, ones_v, zeros_v, acc_sh):
    cid = lax.axis_index("c")
    sid = lax.axis_index("s")
    _fill_f32(ones_v, CH // 16, 1.0)
    _fill_f32(zeros_v, NPT // 16, 0.0)
    pltpu.sync_copy(zeros_v, acc_sh.at[pl.ds(sid * NPT, NPT)])
    plsc.subcore_barrier()

    rowbase = (cid * NS + sid) * (EP // (NC * NS) // CH)
    nblk = EP // (NC * NS) // BLK_E

    def blk(b, _):
        pltpu.sync_copy(dst2d_hbm.at[pl.ds(rowbase + b * BCH, BCH), :],
                        didx_blk)
        for j in range(BCH):
            pltpu.sync_copy(ones_v, acc_sh.at[didx_blk.at[j]], add=True)
        return 0
    lax.fori_loop(0, nblk, blk, 0)

    plsc.subcore_barrier()
    pltpu.sync_copy(acc_sh.at[pl.ds(sid * NPT, NPT)],
                    out_hbm.at[cid, pl.ds(sid * NPT, NPT)])


@functools.partial(
    pl.kernel,
    out_type=jax.ShapeDtypeStruct((NC, NP), jnp.float32),
    mesh=_mesh,
    compiler_params=pltpu.CompilerParams(needs_layout_passes=False, use_tc_tiling_on_sc=False),
    scratch_types=[
        pltpu.VMEM((BCH, CH), jnp.int32),
        pltpu.VMEM((BCH, CH), jnp.int32),
        pltpu.VMEM((BCH, CH), jnp.float32),
        pltpu.VMEM((NP,), jnp.float32),
        pltpu.VMEM((NPT,), jnp.float32),
        pltpu.VMEM_SHARED((NP,), jnp.float32),
    ],
)
def _agg1w_kernel(src2d_hbm, dst2d_hbm, g0_hbm, out_hbm,
                  sidx_blk, didx_blk, stage_blk, table_v, zeros_v, acc_sh):
    cid = lax.axis_index("c")
    sid = lax.axis_index("s")
    _fill_f32(zeros_v, NPT // 16, 0.0)
    pltpu.sync_copy(zeros_v, acc_sh.at[pl.ds(sid * NPT, NPT)])
    pltpu.sync_copy(g0_hbm, table_v)
    plsc.subcore_barrier()

    rowbase = (cid * NS + sid) * (EP // (NC * NS) // CH)
    nblk = EP // (NC * NS) // BLK_E

    def blk(b, _):
        r0 = rowbase + b * BCH
        pltpu.sync_copy(src2d_hbm.at[pl.ds(r0, BCH), :], sidx_blk)
        pltpu.sync_copy(dst2d_hbm.at[pl.ds(r0, BCH), :], didx_blk)
        for j in range(BCH):
            for i in range(CH // 16):
                idx = sidx_blk[j, pl.ds(i * 16, 16)]
                stage_blk[j, pl.ds(i * 16, 16)] = plsc.load_gather(
                    table_v, [idx])
        for j in range(BCH):
            pltpu.sync_copy(stage_blk.at[j], acc_sh.at[didx_blk.at[j]],
                            add=True)
        return 0
    lax.fori_loop(0, nblk, blk, 0)

    plsc.subcore_barrier()
    pltpu.sync_copy(acc_sh.at[pl.ds(sid * NPT, NPT)],
                    out_hbm.at[cid, pl.ds(sid * NPT, NPT)])


@functools.partial(
    pl.kernel,
    out_type=jax.ShapeDtypeStruct((NC, NP, 16), jnp.float32),
    mesh=_mesh,
    compiler_params=pltpu.CompilerParams(needs_layout_passes=False, use_tc_tiling_on_sc=False),
    scratch_types=[
        pltpu.VMEM((BCH, CH), jnp.int32),
        pltpu.VMEM((BCH, CH), jnp.int32),
        pltpu.VMEM((BCH, CH, 16), jnp.float32),
        pltpu.VMEM((NPT // 16, 16), jnp.float32),
        pltpu.VMEM_SHARED((NP, 16), jnp.float32),
        pltpu.SemaphoreType.DMA,
    ],
)
def _agg32_kernel(src2d_hbm, dst2d_hbm, table_hbm, out_hbm,
                  sidx_blk, didx_blk, rows_blk, zeros2d, acc_sh, sem):
    cid = lax.axis_index("c")
    sid = lax.axis_index("s")
    _fill_rows_f32(zeros2d, NPT // 16, 0.0)

    def zb(k, _):
        pltpu.sync_copy(
            zeros2d,
            acc_sh.at[pl.ds(sid * NPT + k * (NPT // 16), NPT // 16), :])
        return 0
    lax.fori_loop(0, NS, zb, 0)
    plsc.subcore_barrier()

    rowbase = sid * (EP // NS // CH)
    nblk = EP // NS // BLK_E
    off = cid * NP

    def blk(b, _):
        r0 = rowbase + b * BCH
        pltpu.sync_copy(src2d_hbm.at[pl.ds(r0, BCH), :], sidx_blk)
        pltpu.sync_copy(dst2d_hbm.at[pl.ds(r0, BCH), :], didx_blk)
        for j in range(BCH):
            for i in range(CH // 16):
                sidx_blk[j, pl.ds(i * 16, 16)] = (
                    sidx_blk[j, pl.ds(i * 16, 16)] + off)
        descs = [
            pltpu.async_copy(table_hbm.at[sidx_blk.at[j]], rows_blk.at[j],
                             sem)
            for j in range(BCH)
        ]
        for d in descs:
            d.wait()
        for j in range(BCH):
            pltpu.sync_copy(rows_blk.at[j], acc_sh.at[didx_blk.at[j]],
                            add=True)
        return 0
    lax.fori_loop(0, nblk, blk, 0)

    plsc.subcore_barrier()
    pltpu.sync_copy(acc_sh.at[pl.ds(sid * NPT, NPT), :],
                    out_hbm.at[cid, pl.ds(sid * NPT, NPT), :])


BN = 2048
GRID = NP // BN


def _k2_body(degp_ref, x_ref, dinv_ref, g0_ref):
    d = degp_ref[0] + degp_ref[1] + 1.0
    dv = lax.rsqrt(jnp.maximum(d, 1e-12))
    dinv_ref[...] = dv
    g0_ref[...] = x_ref[...] * dv


def _k2(degp3, x2):
    return pl.pallas_call(
        _k2_body,
        grid=(NROW // 16,),
        in_specs=[
            pl.BlockSpec((2, 16, 128), lambda i: (0, i, 0)),
            pl.BlockSpec((16, 128), lambda i: (i, 0)),
        ],
        out_specs=[
            pl.BlockSpec((16, 128), lambda i: (i, 0)),
            pl.BlockSpec((16, 128), lambda i: (i, 0)),
        ],
        out_shape=[
            jax.ShapeDtypeStruct((NROW, 128), jnp.float32),
            jax.ShapeDtypeStruct((NROW, 128), jnp.float32),
        ],
    )(degp3, x2)


def _k4_body(dinv_ref, agg_ref, g0_ref, mask_ref, w1_ref, b1_ref, out_ref):
    dv = dinv_ref[...]
    s0 = dv * (agg_ref[0] + agg_ref[1] + g0_ref[...])
    h1 = jnp.maximum(s0 * w1_ref[0:1, :] + b1_ref[0:1, :], 0.0)
    g1 = mask_ref[...] * dv * h1
    out_ref[0] = g1[:, 0:16]
    out_ref[1] = g1[:, 16:32]


def _k4(dinv_c, agg0, g0_c, mask_c, w1p, b1p):
    return pl.pallas_call(
        _k4_body,
        grid=(GRID,),
        in_specs=[
            pl.BlockSpec((BN, 1), lambda i: (i, 0)),
            pl.BlockSpec((2, BN, 1), lambda i: (0, i, 0)),
            pl.BlockSpec((BN, 1), lambda i: (i, 0)),
            pl.BlockSpec((BN, 1), lambda i: (i, 0)),
            pl.BlockSpec((8, HIDDEN), lambda i: (0, 0)),
            pl.BlockSpec((8, HIDDEN), lambda i: (0, 0)),
        ],
        out_specs=pl.BlockSpec((2, BN, 16), lambda i: (0, i, 0)),
        out_shape=jax.ShapeDtypeStruct((2, NP, 16), jnp.float32),
    )(dinv_c, agg0, g0_c, mask_c, w1p, b1p)


def _k6_body(agg_ref, g1_ref, dinv_ref, mask_ref, w2_ref, b2_ref, out_ref):
    a = jnp.concatenate([agg_ref[0], agg_ref[1]], axis=1)
    g = jnp.concatenate([g1_ref[0], g1_ref[1]], axis=1)
    dv = dinv_ref[...]
    s1 = dv * (a + g)
    h2 = jnp.dot(s1, w2_ref[...], preferred_element_type=jnp.float32)
    h2 = jnp.maximum(h2 + b2_ref[0:1, :], 0.0)
    g2 = mask_ref[...] * dv * h2
    out_ref[0] = g2[:, 0:16]
    out_ref[1] = g2[:, 16:32]


def _k6(agg1, g1, dinv_c, mask_c, w2, b2p):
    return pl.pallas_call(
        _k6_body,
        grid=(GRID,),
        in_specs=[
            pl.BlockSpec((2, BN, 16), lambda i: (0, i, 0)),
            pl.BlockSpec((2, BN, 16), lambda i: (0, i, 0)),
            pl.BlockSpec((BN, 1), lambda i: (i, 0)),
            pl.BlockSpec((BN, 1), lambda i: (i, 0)),
            pl.BlockSpec((HIDDEN, HIDDEN), lambda i: (0, 0)),
            pl.BlockSpec((8, HIDDEN), lambda i: (0, 0)),
        ],
        out_specs=pl.BlockSpec((2, BN, 16), lambda i: (0, i, 0)),
        out_shape=jax.ShapeDtypeStruct((2, NP, 16), jnp.float32),
    )(agg1, g1, dinv_c, mask_c, w2, b2p)


def _k8_body(agg_ref, g2_ref, dinv_ref, batch_ref, w3_ref, b3_ref,
             wl_ref, bl_ref, out_ref, sums_ref, cnt_ref):
    i = pl.program_id(0)

    @pl.when(i == 0)
    def _init():
        sums_ref[...] = jnp.zeros((NUM_GRAPHS, HIDDEN), jnp.float32)
        cnt_ref[...] = jnp.zeros((NUM_GRAPHS, 1), jnp.float32)

    a = jnp.concatenate([agg_ref[0], agg_ref[1]], axis=1)
    g = jnp.concatenate([g2_ref[0], g2_ref[1]], axis=1)
    s2 = dinv_ref[...] * (a + g)
    bt = batch_ref[...]
    gids = lax.broadcasted_iota(jnp.int32, (1, NUM_GRAPHS), 1)
    oh = (bt == gids).astype(jnp.float32)
    dn = (((0,), (0,)), ((), ()))
    sums_ref[...] += lax.dot_general(oh, s2, dn,
                                     preferred_element_type=jnp.float32)
    cnt_ref[...] += lax.dot_general(oh, jnp.ones((BN, 1), jnp.float32), dn,
                                    preferred_element_type=jnp.float32)

    @pl.when(i == GRID - 1)
    def _final():
        pool = sums_ref[...] / jnp.maximum(cnt_ref[...], 1.0)
        o1 = jnp.dot(pool, w3_ref[...],
                     preferred_element_type=jnp.float32) + b3_ref[0:1, :]
        o2 = jnp.dot(o1, wl_ref[...],
                     preferred_element_type=jnp.float32) + bl_ref[0:1, :]
        out_ref[...] = o2


def _k8(agg2, g2, dinv_c, batch_c, w3, b3p, wlp, blp):
    return pl.pallas_call(
        _k8_body,
        grid=(GRID,),
        in_specs=[
            pl.BlockSpec((2, BN, 16), lambda i: (0, i, 0)),
            pl.BlockSpec((2, BN, 16), lambda i: (0, i, 0)),
            pl.BlockSpec((BN, 1), lambda i: (i, 0)),
            pl.BlockSpec((BN, 1), lambda i: (i, 0)),
            pl.BlockSpec((HIDDEN, HIDDEN), lambda i: (0, 0)),
            pl.BlockSpec((8, HIDDEN), lambda i: (0, 0)),
            pl.BlockSpec((HIDDEN, 8), lambda i: (0, 0)),
            pl.BlockSpec((8, 8), lambda i: (0, 0)),
        ],
        out_specs=pl.BlockSpec((NUM_GRAPHS, 8), lambda i: (0, 0)),
        out_shape=jax.ShapeDtypeStruct((NUM_GRAPHS, 8), jnp.float32),
        scratch_shapes=[
            pltpu.VMEM((NUM_GRAPHS, HIDDEN), jnp.float32),
            pltpu.VMEM((NUM_GRAPHS, 1), jnp.float32),
        ],
    )(agg2, g2, dinv_c, batch_c, w3, b3p, wlp, blp)


def kernel(x, edge_index, batch, W1, b1, W2, b2, W3, b3, Wl, bl):
    pad_e = EP - E
    srcp = jnp.concatenate(
        [edge_index[0], jnp.full((pad_e,), N, jnp.int32)])
    dstp = jnp.concatenate(
        [edge_index[1], jnp.full((pad_e,), N, jnp.int32)])
    src2d = srcp.reshape(ER2D, CH)
    dst2d = dstp.reshape(ER2D, CH)

    x2 = jnp.pad(x[:, 0], (0, NP - N)).reshape(NROW, 128)
    mask_c = jnp.pad(jnp.ones((N, 1), jnp.float32), ((0, NP - N), (0, 0)))
    batch_c = jnp.pad(batch, (0, NP - N),
                      constant_values=NUM_GRAPHS).reshape(NP, 1)

    w1p = jnp.pad(W1, ((0, 7), (0, 0)))
    b1p = jnp.pad(b1[None, :], ((0, 7), (0, 0)))
    b2p = jnp.pad(b2[None, :], ((0, 7), (0, 0)))
    b3p = jnp.pad(b3[None, :], ((0, 7), (0, 0)))
    wlp = jnp.pad(Wl, ((0, 0), (0, 8 - NUM_CLASSES)))
    blp = jnp.pad(bl[None, :], ((0, 7), (0, 8 - NUM_CLASSES)))

    degp = _deg_kernel(dst2d)
    dinv2, g02 = _k2(degp.reshape(2, NROW, 128), x2)
    dinv_c = dinv2.reshape(NP, 1)
    g0_flat = g02.reshape(NP)

    agg0 = _agg1w_kernel(src2d, dst2d, g0_flat)
    g1 = _k4(dinv_c, agg0.reshape(2, NP, 1),
             g02.reshape(NP, 1), mask_c, w1p, b1p)

    agg1 = _agg32_kernel(src2d, dst2d, g1.reshape(2 * NP, 16))
    g2 = _k6(agg1, g1, dinv_c, mask_c, W2, b2p)

    agg2 = _agg32_kernel(src2d, dst2d, g2.reshape(2 * NP, 16))
    outp = _k8(agg2, g2, dinv_c, batch_c, W3, b3p, wlp, blp)
    return outp[:, :NUM_CLASSES]


# speedup vs baseline: 40.3202x; 1.1359x over previous
import functools

import jax
import jax.numpy as jnp
from jax import lax
from jax.experimental import pallas as pl
from jax.experimental.pallas import tpu as pltpu
from jax.experimental.pallas import tpu_sc as plsc

N = 100000
E = 3200000
NUM_GRAPHS = 64
HIDDEN = 32
NUM_CLASSES = 3

NP = 100352
NROW = NP // 128
EP = 3211264
NC = 2
NS = 16
CH = 128
BCH = 8
BLK_E = CH * BCH
NPT = NP // NS
ER2D = EP // CH

_mesh = plsc.VectorSubcoreMesh(core_axis_name="c", subcore_axis_name="s")


def _fill_f32(buf, n16, value):
    def body(i, _):
        buf[pl.ds(i * 16, 16)] = jnp.full((16,), value, jnp.float32)
        return 0
    lax.fori_loop(0, n16, body, 0)


def _fill_rows_f32(buf, nrows, value):
    def body(i, _):
        buf[i, :] = jnp.full((16,), value, jnp.float32)
        return 0
    lax.fori_loop(0, nrows, body, 0)


@functools.partial(
    pl.kernel,
    out_type=jax.ShapeDtypeStruct((NC, NP), jnp.float32),
    mesh=_mesh,
    compiler_params=pltpu.CompilerParams(needs_layout_passes=False, use_tc_tiling_on_sc=False),
    scratch_types=[
        pltpu.VMEM((BCH, CH), jnp.int32),
        pltpu.VMEM((CH,), jnp.float32),
        pltpu.VMEM((NPT,), jnp.float32),
        pltpu.VMEM_SHARED((NP,), jnp.float32),
    ],
)
def _deg_kernel(dst2d_hbm, out_hbm, didx_blk, ones_v, zeros_v, acc_sh):
    cid = lax.axis_index("c")
    sid = lax.axis_index("s")
    _fill_f32(ones_v, CH // 16, 1.0)
    _fill_f32(zeros_v, NPT // 16, 0.0)
    pltpu.sync_copy(zeros_v, acc_sh.at[pl.ds(sid * NPT, NPT)])
    plsc.subcore_barrier()

    rowbase = (cid * NS + sid) * (EP // (NC * NS) // CH)
    nblk = EP // (NC * NS) // BLK_E

    def blk(b, _):
        pltpu.sync_copy(dst2d_hbm.at[pl.ds(rowbase + b * BCH, BCH), :],
                        didx_blk)
        for j in range(BCH):
            pltpu.sync_copy(ones_v, acc_sh.at[didx_blk.at[j]], add=True)
        return 0
    lax.fori_loop(0, nblk, blk, 0)

    plsc.subcore_barrier()
    pltpu.sync_copy(acc_sh.at[pl.ds(sid * NPT, NPT)],
                    out_hbm.at[cid, pl.ds(sid * NPT, NPT)])


@functools.partial(
    pl.kernel,
    out_type=jax.ShapeDtypeStruct((NC, NP), jnp.float32),
    mesh=_mesh,
    compiler_params=pltpu.CompilerParams(needs_layout_passes=False, use_tc_tiling_on_sc=False),
    scratch_types=[
        pltpu.VMEM((BCH, CH), jnp.int32),
        pltpu.VMEM((BCH, CH), jnp.int32),
        pltpu.VMEM((BCH, CH), jnp.float32),
        pltpu.VMEM((NP,), jnp.float32),
        pltpu.VMEM((NPT,), jnp.float32),
        pltpu.VMEM_SHARED((NP,), jnp.float32),
    ],
)
def _agg1w_kernel(src2d_hbm, dst2d_hbm, g0_hbm, out_hbm,
                  sidx_blk, didx_blk, stage_blk, table_v, zeros_v, acc_sh):
    cid = lax.axis_index("c")
    sid = lax.axis_index("s")
    _fill_f32(zeros_v, NPT // 16, 0.0)
    pltpu.sync_copy(zeros_v, acc_sh.at[pl.ds(sid * NPT, NPT)])
    pltpu.sync_copy(g0_hbm, table_v)
    plsc.subcore_barrier()

    rowbase = (cid * NS + sid) * (EP // (NC * NS) // CH)
    nblk = EP // (NC * NS) // BLK_E

    def blk(b, _):
        r0 = rowbase + b * BCH
        pltpu.sync_copy(src2d_hbm.at[pl.ds(r0, BCH), :], sidx_blk)
        pltpu.sync_copy(dst2d_hbm.at[pl.ds(r0, BCH), :], didx_blk)
        for j in range(BCH):
            for i in range(CH // 16):
                idx = sidx_blk[j, pl.ds(i * 16, 16)]
                stage_blk[j, pl.ds(i * 16, 16)] = plsc.load_gather(
                    table_v, [idx])
        for j in range(BCH):
            pltpu.sync_copy(stage_blk.at[j], acc_sh.at[didx_blk.at[j]],
                            add=True)
        return 0
    lax.fori_loop(0, nblk, blk, 0)

    plsc.subcore_barrier()
    pltpu.sync_copy(acc_sh.at[pl.ds(sid * NPT, NPT)],
                    out_hbm.at[cid, pl.ds(sid * NPT, NPT)])


@functools.partial(
    pl.kernel,
    out_type=jax.ShapeDtypeStruct((NC, NP, 16), jnp.float32),
    mesh=_mesh,
    compiler_params=pltpu.CompilerParams(needs_layout_passes=False, use_tc_tiling_on_sc=False),
    scratch_types=[
        pltpu.VMEM((BCH, CH), jnp.int32),
        pltpu.VMEM((BCH, CH), jnp.int32),
        pltpu.VMEM((BCH, CH, 16), jnp.float32),
        pltpu.VMEM((49, 16), jnp.float32),
        pltpu.VMEM_SHARED((NP, 16), jnp.float32),
        pltpu.SemaphoreType.DMA,
        pltpu.SemaphoreType.DMA,
    ],
)
def _agg32_kernel(src2d_hbm, dst2d_hbm, table_hbm, out_hbm,
                  sidx_blk, didx_blk, rows_blk, zeros2d, acc_sh, gsem, ssem):
    cid = lax.axis_index("c")
    sid = lax.axis_index("s")
    _fill_rows_f32(zeros2d, 49, 0.0)

    def zb(k, _):
        pltpu.sync_copy(zeros2d,
                        acc_sh.at[pl.ds(sid * NPT + k * 49, 49), :])
        return 0
    lax.fori_loop(0, NPT // 49, zb, 0)
    plsc.subcore_barrier()

    rowbase = sid * (EP // NS // CH)
    nblk = EP // NS // BLK_E
    off = cid * NP
    HB = BCH // 2

    def blk(t, _):
        r0 = rowbase + t * BCH
        pltpu.sync_copy(src2d_hbm.at[pl.ds(r0, BCH), :], sidx_blk)
        pltpu.sync_copy(dst2d_hbm.at[pl.ds(r0, BCH), :], didx_blk)
        for j in range(BCH):
            for i in range(CH // 16):
                sidx_blk[j, pl.ds(i * 16, 16)] = (
                    sidx_blk[j, pl.ds(i * 16, 16)] + off)
        descs = [
            pltpu.async_copy(table_hbm.at[sidx_blk.at[j]], rows_blk.at[j],
                             gsem)
            for j in range(BCH)
        ]
        sdescs = []
        for j in range(HB):
            descs[j].wait()
            sdescs.append(pltpu.make_async_copy(
                rows_blk.at[j], acc_sh.at[didx_blk.at[j]], ssem))
            sdescs[-1].start(add=True)
        for j in range(HB, BCH):
            descs[j].wait()
            sdescs.append(pltpu.make_async_copy(
                rows_blk.at[j], acc_sh.at[didx_blk.at[j]], ssem))
            sdescs[-1].start(add=True)
        for d in sdescs:
            d.wait()
        return 0
    lax.fori_loop(0, nblk, blk, 0)

    plsc.subcore_barrier()
    pltpu.sync_copy(acc_sh.at[pl.ds(sid * NPT, NPT), :],
                    out_hbm.at[cid, pl.ds(sid * NPT, NPT), :])


BN = 2048
GRID = NP // BN


def _k2_body(degp_ref, x_ref, dinv_ref, g0_ref):
    d = degp_ref[0] + degp_ref[1] + 1.0
    dv = lax.rsqrt(jnp.maximum(d, 1e-12))
    dinv_ref[...] = dv
    g0_ref[...] = x_ref[...] * dv


def _k2(degp3, x2):
    return pl.pallas_call(
        _k2_body,
        grid=(NROW // 16,),
        in_specs=[
            pl.BlockSpec((2, 16, 128), lambda i: (0, i, 0)),
            pl.BlockSpec((16, 128), lambda i: (i, 0)),
        ],
        out_specs=[
            pl.BlockSpec((16, 128), lambda i: (i, 0)),
            pl.BlockSpec((16, 128), lambda i: (i, 0)),
        ],
        out_shape=[
            jax.ShapeDtypeStruct((NROW, 128), jnp.float32),
            jax.ShapeDtypeStruct((NROW, 128), jnp.float32),
        ],
    )(degp3, x2)


def _k4_body(dinv_ref, agg_ref, g0_ref, mask_ref, w1_ref, b1_ref, out_ref):
    dv = dinv_ref[...]
    s0 = dv * (agg_ref[0] + agg_ref[1] + g0_ref[...])
    h1 = jnp.maximum(s0 * w1_ref[0:1, :] + b1_ref[0:1, :], 0.0)
    g1 = mask_ref[...] * dv * h1
    out_ref[0] = g1[:, 0:16]
    out_ref[1] = g1[:, 16:32]


def _k4(dinv_c, agg0, g0_c, mask_c, w1p, b1p):
    return pl.pallas_call(
        _k4_body,
        grid=(GRID,),
        in_specs=[
            pl.BlockSpec((BN, 1), lambda i: (i, 0)),
            pl.BlockSpec((2, BN, 1), lambda i: (0, i, 0)),
            pl.BlockSpec((BN, 1), lambda i: (i, 0)),
            pl.BlockSpec((BN, 1), lambda i: (i, 0)),
            pl.BlockSpec((8, HIDDEN), lambda i: (0, 0)),
            pl.BlockSpec((8, HIDDEN), lambda i: (0, 0)),
        ],
        out_specs=pl.BlockSpec((2, BN, 16), lambda i: (0, i, 0)),
        out_shape=jax.ShapeDtypeStruct((2, NP, 16), jnp.float32),
    )(dinv_c, agg0, g0_c, mask_c, w1p, b1p)


def _k6_body(agg_ref, g1_ref, dinv_ref, mask_ref, w2_ref, b2_ref, out_ref):
    a = jnp.concatenate([agg_ref[0], agg_ref[1]], axis=1)
    g = jnp.concatenate([g1_ref[0], g1_ref[1]], axis=1)
    dv = dinv_ref[...]
    s1 = dv * (a + g)
    h2 = jnp.dot(s1, w2_ref[...], preferred_element_type=jnp.float32)
    h2 = jnp.maximum(h2 + b2_ref[0:1, :], 0.0)
    g2 = mask_ref[...] * dv * h2
    out_ref[0] = g2[:, 0:16]
    out_ref[1] = g2[:, 16:32]


def _k6(agg1, g1, dinv_c, mask_c, w2, b2p):
    return pl.pallas_call(
        _k6_body,
        grid=(GRID,),
        in_specs=[
            pl.BlockSpec((2, BN, 16), lambda i: (0, i, 0)),
            pl.BlockSpec((2, BN, 16), lambda i: (0, i, 0)),
            pl.BlockSpec((BN, 1), lambda i: (i, 0)),
            pl.BlockSpec((BN, 1), lambda i: (i, 0)),
            pl.BlockSpec((HIDDEN, HIDDEN), lambda i: (0, 0)),
            pl.BlockSpec((8, HIDDEN), lambda i: (0, 0)),
        ],
        out_specs=pl.BlockSpec((2, BN, 16), lambda i: (0, i, 0)),
        out_shape=jax.ShapeDtypeStruct((2, NP, 16), jnp.float32),
    )(agg1, g1, dinv_c, mask_c, w2, b2p)


def _k8_body(agg_ref, g2_ref, dinv_ref, batch_ref, w3_ref, b3_ref,
             wl_ref, bl_ref, out_ref, sums_ref, cnt_ref):
    i = pl.program_id(0)

    @pl.when(i == 0)
    def _init():
        sums_ref[...] = jnp.zeros((NUM_GRAPHS, HIDDEN), jnp.float32)
        cnt_ref[...] = jnp.zeros((NUM_GRAPHS, 1), jnp.float32)

    a = jnp.concatenate([agg_ref[0], agg_ref[1]], axis=1)
    g = jnp.concatenate([g2_ref[0], g2_ref[1]], axis=1)
    s2 = dinv_ref[...] * (a + g)
    bt = batch_ref[...]
    gids = lax.broadcasted_iota(jnp.int32, (1, NUM_GRAPHS), 1)
    oh = (bt == gids).astype(jnp.float32)
    dn = (((0,), (0,)), ((), ()))
    sums_ref[...] += lax.dot_general(oh, s2, dn,
                                     preferred_element_type=jnp.float32)
    cnt_ref[...] += lax.dot_general(oh, jnp.ones((BN, 1), jnp.float32), dn,
                                    preferred_element_type=jnp.float32)

    @pl.when(i == GRID - 1)
    def _final():
        pool = sums_ref[...] / jnp.maximum(cnt_ref[...], 1.0)
        o1 = jnp.dot(pool, w3_ref[...],
                     preferred_element_type=jnp.float32) + b3_ref[0:1, :]
        o2 = jnp.dot(o1, wl_ref[...],
                     preferred_element_type=jnp.float32) + bl_ref[0:1, :]
        out_ref[...] = o2


def _k8(agg2, g2, dinv_c, batch_c, w3, b3p, wlp, blp):
    return pl.pallas_call(
        _k8_body,
        grid=(GRID,),
        in_specs=[
            pl.BlockSpec((2, BN, 16), lambda i: (0, i, 0)),
            pl.BlockSpec((2, BN, 16), lambda i: (0, i, 0)),
            pl.BlockSpec((BN, 1), lambda i: (i, 0)),
            pl.BlockSpec((BN, 1), lambda i: (i, 0)),
            pl.BlockSpec((HIDDEN, HIDDEN), lambda i: (0, 0)),
            pl.BlockSpec((8, HIDDEN), lambda i: (0, 0)),
            pl.BlockSpec((HIDDEN, 8), lambda i: (0, 0)),
            pl.BlockSpec((8, 8), lambda i: (0, 0)),
        ],
        out_specs=pl.BlockSpec((NUM_GRAPHS, 8), lambda i: (0, 0)),
        out_shape=jax.ShapeDtypeStruct((NUM_GRAPHS, 8), jnp.float32),
        scratch_shapes=[
            pltpu.VMEM((NUM_GRAPHS, HIDDEN), jnp.float32),
            pltpu.VMEM((NUM_GRAPHS, 1), jnp.float32),
        ],
    )(agg2, g2, dinv_c, batch_c, w3, b3p, wlp, blp)


def kernel(x, edge_index, batch, W1, b1, W2, b2, W3, b3, Wl, bl):
    pad_e = EP - E
    srcp = jnp.concatenate(
        [edge_index[0], jnp.full((pad_e,), N, jnp.int32)])
    dstp = jnp.concatenate(
        [edge_index[1], jnp.full((pad_e,), N, jnp.int32)])
    src2d = srcp.reshape(ER2D, CH)
    dst2d = dstp.reshape(ER2D, CH)

    x2 = jnp.pad(x[:, 0], (0, NP - N)).reshape(NROW, 128)
    mask_c = jnp.pad(jnp.ones((N, 1), jnp.float32), ((0, NP - N), (0, 0)))
    batch_c = jnp.pad(batch, (0, NP - N),
                      constant_values=NUM_GRAPHS).reshape(NP, 1)

    w1p = jnp.pad(W1, ((0, 7), (0, 0)))
    b1p = jnp.pad(b1[None, :], ((0, 7), (0, 0)))
    b2p = jnp.pad(b2[None, :], ((0, 7), (0, 0)))
    b3p = jnp.pad(b3[None, :], ((0, 7), (0, 0)))
    wlp = jnp.pad(Wl, ((0, 0), (0, 8 - NUM_CLASSES)))
    blp = jnp.pad(bl[None, :], ((0, 7), (0, 8 - NUM_CLASSES)))

    degp = _deg_kernel(dst2d)
    dinv2, g02 = _k2(degp.reshape(2, NROW, 128), x2)
    dinv_c = dinv2.reshape(NP, 1)
    g0_flat = g02.reshape(NP)

    agg0 = _agg1w_kernel(src2d, dst2d, g0_flat)
    g1 = _k4(dinv_c, agg0.reshape(2, NP, 1),
             g02.reshape(NP, 1), mask_c, w1p, b1p)

    agg1 = _agg32_kernel(src2d, dst2d, g1.reshape(2 * NP, 16))
    g2 = _k6(agg1, g1, dinv_c, mask_c, W2, b2p)

    agg2 = _agg32_kernel(src2d, dst2d, g2.reshape(2 * NP, 16))
    outp = _k8(agg2, g2, dinv_c, batch_c, W3, b3p, wlp, blp)
    return outp[:, :NUM_CLASSES]


# speedup vs baseline: 44.0068x; 1.0914x over previous
import functools

import jax
import jax.numpy as jnp
from jax import lax
from jax.experimental import pallas as pl
from jax.experimental.pallas import tpu as pltpu
from jax.experimental.pallas import tpu_sc as plsc

N = 100000
E = 3200000
NUM_GRAPHS = 64
HIDDEN = 32
NUM_CLASSES = 3

NP = 100352
NROW = NP // 128
EP = 3211264
NC = 2
NS = 16
CH = 128
BCH = 8
BLK_E = CH * BCH
NPT = NP // NS
ER2D = EP // CH

_mesh = plsc.VectorSubcoreMesh(core_axis_name="c", subcore_axis_name="s")


def _fill_f32(buf, n16, value):
    def body(i, _):
        buf[pl.ds(i * 16, 16)] = jnp.full((16,), value, jnp.float32)
        return 0
    lax.fori_loop(0, n16, body, 0)


def _fill_rows_f32(buf, nrows, value):
    def body(i, _):
        buf[i, :] = jnp.full((16,), value, jnp.float32)
        return 0
    lax.fori_loop(0, nrows, body, 0)


@functools.partial(
    pl.kernel,
    out_type=jax.ShapeDtypeStruct((NC, NP), jnp.float32),
    mesh=_mesh,
    compiler_params=pltpu.CompilerParams(needs_layout_passes=False, use_tc_tiling_on_sc=False),
    scratch_types=[
        pltpu.VMEM((BCH, CH), jnp.int32),
        pltpu.VMEM((CH,), jnp.float32),
        pltpu.VMEM((NPT,), jnp.float32),
        pltpu.VMEM_SHARED((NP,), jnp.float32),
    ],
)
def _deg_kernel(dst2d_hbm, out_hbm, didx_blk, ones_v, zeros_v, acc_sh):
    cid = lax.axis_index("c")
    sid = lax.axis_index("s")
    _fill_f32(ones_v, CH // 16, 1.0)
    _fill_f32(zeros_v, NPT // 16, 0.0)
    pltpu.sync_copy(zeros_v, acc_sh.at[pl.ds(sid * NPT, NPT)])
    plsc.subcore_barrier()

    rowbase = (cid * NS + sid) * (EP // (NC * NS) // CH)
    nblk = EP // (NC * NS) // BLK_E

    def blk(b, _):
        pltpu.sync_copy(dst2d_hbm.at[pl.ds(rowbase + b * BCH, BCH), :],
                        didx_blk)
        for j in range(BCH):
            pltpu.sync_copy(ones_v, acc_sh.at[didx_blk.at[j]], add=True)
        return 0
    lax.fori_loop(0, nblk, blk, 0)

    plsc.subcore_barrier()
    pltpu.sync_copy(acc_sh.at[pl.ds(sid * NPT, NPT)],
                    out_hbm.at[cid, pl.ds(sid * NPT, NPT)])


@functools.partial(
    pl.kernel,
    out_type=jax.ShapeDtypeStruct((NC, NP), jnp.float32),
    mesh=_mesh,
    compiler_params=pltpu.CompilerParams(needs_layout_passes=False, use_tc_tiling_on_sc=False),
    scratch_types=[
        pltpu.VMEM((BCH, CH), jnp.int32),
        pltpu.VMEM((BCH, CH), jnp.int32),
        pltpu.VMEM((BCH, CH), jnp.float32),
        pltpu.VMEM((NP,), jnp.float32),
        pltpu.VMEM((NPT,), jnp.float32),
        pltpu.VMEM_SHARED((NP,), jnp.float32),
    ],
)
def _agg1w_kernel(src2d_hbm, dst2d_hbm, g0_hbm, out_hbm,
                  sidx_blk, didx_blk, stage_blk, table_v, zeros_v, acc_sh):
    cid = lax.axis_index("c")
    sid = lax.axis_index("s")
    _fill_f32(zeros_v, NPT // 16, 0.0)
    pltpu.sync_copy(zeros_v, acc_sh.at[pl.ds(sid * NPT, NPT)])
    pltpu.sync_copy(g0_hbm, table_v)
    plsc.subcore_barrier()

    rowbase = (cid * NS + sid) * (EP // (NC * NS) // CH)
    nblk = EP // (NC * NS) // BLK_E

    def blk(b, _):
        r0 = rowbase + b * BCH
        pltpu.sync_copy(src2d_hbm.at[pl.ds(r0, BCH), :], sidx_blk)
        pltpu.sync_copy(dst2d_hbm.at[pl.ds(r0, BCH), :], didx_blk)
        for j in range(BCH):
            for i in range(CH // 16):
                idx = sidx_blk[j, pl.ds(i * 16, 16)]
                stage_blk[j, pl.ds(i * 16, 16)] = plsc.load_gather(
                    table_v, [idx])
        for j in range(BCH):
            pltpu.sync_copy(stage_blk.at[j], acc_sh.at[didx_blk.at[j]],
                            add=True)
        return 0
    lax.fori_loop(0, nblk, blk, 0)

    plsc.subcore_barrier()
    pltpu.sync_copy(acc_sh.at[pl.ds(sid * NPT, NPT)],
                    out_hbm.at[cid, pl.ds(sid * NPT, NPT)])


@functools.partial(
    pl.kernel,
    out_type=jax.ShapeDtypeStruct((NC, NP, 16), jnp.float32),
    mesh=_mesh,
    compiler_params=pltpu.CompilerParams(needs_layout_passes=False, use_tc_tiling_on_sc=False),
    scratch_types=[
        pltpu.VMEM((4, CH), jnp.int32),
        pltpu.VMEM((4, CH), jnp.int32),
        pltpu.VMEM((4, CH, 16), jnp.float32),
        pltpu.VMEM((4, CH), jnp.int32),
        pltpu.VMEM((4, CH), jnp.int32),
        pltpu.VMEM((4, CH, 16), jnp.float32),
        pltpu.VMEM((49, 16), jnp.float32),
        pltpu.VMEM_SHARED((NP, 16), jnp.float32),
        pltpu.SemaphoreType.DMA,
        pltpu.SemaphoreType.DMA,
        pltpu.SemaphoreType.DMA,
    ],
)
def _agg32_kernel(src2d_hbm, dst2d_hbm, table_hbm, out_hbm,
                  sidxA, didxA, rowsA, sidxB, didxB, rowsB,
                  zeros2d, acc_sh, isem, gsem, ssem):
    cid = lax.axis_index("c")
    sid = lax.axis_index("s")
    _fill_rows_f32(zeros2d, 49, 0.0)

    def zb(k, _):
        pltpu.sync_copy(zeros2d,
                        acc_sh.at[pl.ds(sid * NPT + k * 49, 49), :])
        return 0
    lax.fori_loop(0, NPT // 49, zb, 0)
    plsc.subcore_barrier()

    rowbase = sid * (EP // NS // CH)
    nset = EP // NS // CH // 4
    niter = nset // 2
    off = cid * NP
    last_r = rowbase + (nset - 1) * 4

    def _fire_idx(setblk, sidx, didx):
        r = lax.min(rowbase + setblk * 4, last_r)
        pltpu.async_copy(src2d_hbm.at[pl.ds(r, 4), :], sidx, isem)
        pltpu.async_copy(dst2d_hbm.at[pl.ds(r, 4), :], didx, isem)

    def _wait_idx(sidx, didx):
        pltpu.make_async_copy(src2d_hbm.at[pl.ds(rowbase, 4), :], sidx,
                              isem).wait()
        pltpu.make_async_copy(dst2d_hbm.at[pl.ds(rowbase, 4), :], didx,
                              isem).wait()

    def _drain_scat(rows, didx):
        for j in range(4):
            pltpu.make_async_copy(rows.at[j], acc_sh.at[didx.at[j]],
                                  ssem).wait()

    def _phase(sidx, didx, rows, o_sidx, o_didx, o_rows, pf_blk,
               drain_other):
        _wait_idx(sidx, didx)
        for j in range(4):
            for i in range(CH // 16):
                sidx[j, pl.ds(i * 16, 16)] = sidx[j, pl.ds(i * 16, 16)] + off
        gd = [pltpu.async_copy(table_hbm.at[sidx.at[j]], rows.at[j], gsem)
              for j in range(4)]
        if drain_other is None:
            _drain_scat(o_rows, o_didx)
        else:
            @pl.when(drain_other)
            def _():
                _drain_scat(o_rows, o_didx)
        _fire_idx(pf_blk, o_sidx, o_didx)
        for j in range(4):
            gd[j].wait()
            pltpu.async_copy(rows.at[j], acc_sh.at[didx.at[j]], ssem,
                             add=True)

    _fire_idx(0, sidxA, didxA)

    def blk(t, _):
        _phase(sidxA, didxA, rowsA, sidxB, didxB, rowsB, 2 * t + 1, t > 0)
        _phase(sidxB, didxB, rowsB, sidxA, didxA, rowsA, 2 * t + 2, None)
        return 0
    lax.fori_loop(0, niter, blk, 0)

    _drain_scat(rowsB, didxB)
    _wait_idx(sidxA, didxA)

    plsc.subcore_barrier()
    pltpu.sync_copy(acc_sh.at[pl.ds(sid * NPT, NPT), :],
                    out_hbm.at[cid, pl.ds(sid * NPT, NPT), :])


BN = 2048
GRID = NP // BN


def _k2_body(degp_ref, x_ref, dinv_ref, g0_ref):
    d = degp_ref[0] + degp_ref[1] + 1.0
    dv = lax.rsqrt(jnp.maximum(d, 1e-12))
    dinv_ref[...] = dv
    g0_ref[...] = x_ref[...] * dv


def _k2(degp3, x2):
    return pl.pallas_call(
        _k2_body,
        grid=(NROW // 16,),
        in_specs=[
            pl.BlockSpec((2, 16, 128), lambda i: (0, i, 0)),
            pl.BlockSpec((16, 128), lambda i: (i, 0)),
        ],
        out_specs=[
            pl.BlockSpec((16, 128), lambda i: (i, 0)),
            pl.BlockSpec((16, 128), lambda i: (i, 0)),
        ],
        out_shape=[
            jax.ShapeDtypeStruct((NROW, 128), jnp.float32),
            jax.ShapeDtypeStruct((NROW, 128), jnp.float32),
        ],
    )(degp3, x2)


def _k4_body(dinv_ref, agg_ref, g0_ref, mask_ref, w1_ref, b1_ref, out_ref):
    dv = dinv_ref[...]
    s0 = dv * (agg_ref[0] + agg_ref[1] + g0_ref[...])
    h1 = jnp.maximum(s0 * w1_ref[0:1, :] + b1_ref[0:1, :], 0.0)
    g1 = mask_ref[...] * dv * h1
    out_ref[0] = g1[:, 0:16]
    out_ref[1] = g1[:, 16:32]


def _k4(dinv_c, agg0, g0_c, mask_c, w1p, b1p):
    return pl.pallas_call(
        _k4_body,
        grid=(GRID,),
        in_specs=[
            pl.BlockSpec((BN, 1), lambda i: (i, 0)),
            pl.BlockSpec((2, BN, 1), lambda i: (0, i, 0)),
            pl.BlockSpec((BN, 1), lambda i: (i, 0)),
            pl.BlockSpec((BN, 1), lambda i: (i, 0)),
            pl.BlockSpec((8, HIDDEN), lambda i: (0, 0)),
            pl.BlockSpec((8, HIDDEN), lambda i: (0, 0)),
        ],
        out_specs=pl.BlockSpec((2, BN, 16), lambda i: (0, i, 0)),
        out_shape=jax.ShapeDtypeStruct((2, NP, 16), jnp.float32),
    )(dinv_c, agg0, g0_c, mask_c, w1p, b1p)


def _k6_body(agg_ref, g1_ref, dinv_ref, mask_ref, w2_ref, b2_ref, out_ref):
    a = jnp.concatenate([agg_ref[0], agg_ref[1]], axis=1)
    g = jnp.concatenate([g1_ref[0], g1_ref[1]], axis=1)
    dv = dinv_ref[...]
    s1 = dv * (a + g)
    h2 = jnp.dot(s1, w2_ref[...], preferred_element_type=jnp.float32)
    h2 = jnp.maximum(h2 + b2_ref[0:1, :], 0.0)
    g2 = mask_ref[...] * dv * h2
    out_ref[0] = g2[:, 0:16]
    out_ref[1] = g2[:, 16:32]


def _k6(agg1, g1, dinv_c, mask_c, w2, b2p):
    return pl.pallas_call(
        _k6_body,
        grid=(GRID,),
        in_specs=[
            pl.BlockSpec((2, BN, 16), lambda i: (0, i, 0)),
            pl.BlockSpec((2, BN, 16), lambda i: (0, i, 0)),
            pl.BlockSpec((BN, 1), lambda i: (i, 0)),
            pl.BlockSpec((BN, 1), lambda i: (i, 0)),
            pl.BlockSpec((HIDDEN, HIDDEN), lambda i: (0, 0)),
            pl.BlockSpec((8, HIDDEN), lambda i: (0, 0)),
        ],
        out_specs=pl.BlockSpec((2, BN, 16), lambda i: (0, i, 0)),
        out_shape=jax.ShapeDtypeStruct((2, NP, 16), jnp.float32),
    )(agg1, g1, dinv_c, mask_c, w2, b2p)


def _k8_body(agg_ref, g2_ref, dinv_ref, batch_ref, w3_ref, b3_ref,
             wl_ref, bl_ref, out_ref, sums_ref, cnt_ref):
    i = pl.program_id(0)

    @pl.when(i == 0)
    def _init():
        sums_ref[...] = jnp.zeros((NUM_GRAPHS, HIDDEN), jnp.float32)
        cnt_ref[...] = jnp.zeros((NUM_GRAPHS, 1), jnp.float32)

    a = jnp.concatenate([agg_ref[0], agg_ref[1]], axis=1)
    g = jnp.concatenate([g2_ref[0], g2_ref[1]], axis=1)
    s2 = dinv_ref[...] * (a + g)
    bt = batch_ref[...]
    gids = lax.broadcasted_iota(jnp.int32, (1, NUM_GRAPHS), 1)
    oh = (bt == gids).astype(jnp.float32)
    dn = (((0,), (0,)), ((), ()))
    sums_ref[...] += lax.dot_general(oh, s2, dn,
                                     preferred_element_type=jnp.float32)
    cnt_ref[...] += lax.dot_general(oh, jnp.ones((BN, 1), jnp.float32), dn,
                                    preferred_element_type=jnp.float32)

    @pl.when(i == GRID - 1)
    def _final():
        pool = sums_ref[...] / jnp.maximum(cnt_ref[...], 1.0)
        o1 = jnp.dot(pool, w3_ref[...],
                     preferred_element_type=jnp.float32) + b3_ref[0:1, :]
        o2 = jnp.dot(o1, wl_ref[...],
                     preferred_element_type=jnp.float32) + bl_ref[0:1, :]
        out_ref[...] = o2


def _k8(agg2, g2, dinv_c, batch_c, w3, b3p, wlp, blp):
    return pl.pallas_call(
        _k8_body,
        grid=(GRID,),
        in_specs=[
            pl.BlockSpec((2, BN, 16), lambda i: (0, i, 0)),
            pl.BlockSpec((2, BN, 16), lambda i: (0, i, 0)),
            pl.BlockSpec((BN, 1), lambda i: (i, 0)),
            pl.BlockSpec((BN, 1), lambda i: (i, 0)),
            pl.BlockSpec((HIDDEN, HIDDEN), lambda i: (0, 0)),
            pl.BlockSpec((8, HIDDEN), lambda i: (0, 0)),
            pl.BlockSpec((HIDDEN, 8), lambda i: (0, 0)),
            pl.BlockSpec((8, 8), lambda i: (0, 0)),
        ],
        out_specs=pl.BlockSpec((NUM_GRAPHS, 8), lambda i: (0, 0)),
        out_shape=jax.ShapeDtypeStruct((NUM_GRAPHS, 8), jnp.float32),
        scratch_shapes=[
            pltpu.VMEM((NUM_GRAPHS, HIDDEN), jnp.float32),
            pltpu.VMEM((NUM_GRAPHS, 1), jnp.float32),
        ],
    )(agg2, g2, dinv_c, batch_c, w3, b3p, wlp, blp)


def kernel(x, edge_index, batch, W1, b1, W2, b2, W3, b3, Wl, bl):
    pad_e = EP - E
    srcp = jnp.concatenate(
        [edge_index[0], jnp.full((pad_e,), N, jnp.int32)])
    dstp = jnp.concatenate(
        [edge_index[1], jnp.full((pad_e,), N, jnp.int32)])
    src2d = srcp.reshape(ER2D, CH)
    dst2d = dstp.reshape(ER2D, CH)

    x2 = jnp.pad(x[:, 0], (0, NP - N)).reshape(NROW, 128)
    mask_c = jnp.pad(jnp.ones((N, 1), jnp.float32), ((0, NP - N), (0, 0)))
    batch_c = jnp.pad(batch, (0, NP - N),
                      constant_values=NUM_GRAPHS).reshape(NP, 1)

    w1p = jnp.pad(W1, ((0, 7), (0, 0)))
    b1p = jnp.pad(b1[None, :], ((0, 7), (0, 0)))
    b2p = jnp.pad(b2[None, :], ((0, 7), (0, 0)))
    b3p = jnp.pad(b3[None, :], ((0, 7), (0, 0)))
    wlp = jnp.pad(Wl, ((0, 0), (0, 8 - NUM_CLASSES)))
    blp = jnp.pad(bl[None, :], ((0, 7), (0, 8 - NUM_CLASSES)))

    degp = _deg_kernel(dst2d)
    dinv2, g02 = _k2(degp.reshape(2, NROW, 128), x2)
    dinv_c = dinv2.reshape(NP, 1)
    g0_flat = g02.reshape(NP)

    agg0 = _agg1w_kernel(src2d, dst2d, g0_flat)
    g1 = _k4(dinv_c, agg0.reshape(2, NP, 1),
             g02.reshape(NP, 1), mask_c, w1p, b1p)

    agg1 = _agg32_kernel(src2d, dst2d, g1.reshape(2 * NP, 16))
    g2 = _k6(agg1, g1, dinv_c, mask_c, W2, b2p)

    agg2 = _agg32_kernel(src2d, dst2d, g2.reshape(2 * NP, 16))
    outp = _k8(agg2, g2, dinv_c, batch_c, W3, b3p, wlp, blp)
    return outp[:, :NUM_CLASSES]
